# Initial kernel scaffold; baseline (speedup 1.0000x reference)
import jax
import jax.numpy as jnp
from jax.experimental import pallas as pl


def kernel(x, edge_index, W0, b0, gw0, gb0, ga0, W1, b1, gw1, gb1, ga1, W2, b2, gw2, gb2, ga2):
    raise NotImplementedError("write your pallas kernel here")


# speedup vs baseline: 19.9105x; 19.9105x over previous
import functools

import jax
import jax.numpy as jnp
from jax import lax
from jax.experimental import pallas as pl
from jax.experimental.pallas import tpu as pltpu
from jax.experimental.pallas import tpu_sc as plsc

N = 10000
E = 320000
D_IN = 128
D_H = 64

NC = 2
NS = 16
NW = NC * NS
EPW = E // NW
K = 80
NJ = EPW // K
RPT = 640
L = 16

_mesh = plsc.VectorSubcoreMesh(core_axis_name="c", subcore_axis_name="s")
_sc_params = pltpu.CompilerParams(use_tc_tiling_on_sc=False)


@functools.partial(
    pl.kernel,
    out_type=jax.ShapeDtypeStruct((NC, N, L), jnp.float32),
    mesh=_mesh,
    compiler_params=_sc_params,
    scratch_types=[
        pltpu.VMEM((NJ, K), jnp.int32),
        pltpu.VMEM((K, L), jnp.float32),
        pltpu.VMEM_SHARED((N, L), jnp.float32),
    ],
)
def _deg_kernel(dst_hbm, out_hbm, dst_v, ones_v, acc):
    c = lax.axis_index("c")
    s = lax.axis_index("s")
    w = s * NC + c
    pltpu.sync_copy(dst_hbm.at[w], dst_v)

    def fill(i, carry):
        ones_v[i, :] = jnp.full((L,), carry, jnp.float32)
        return carry

    lax.fori_loop(0, K, fill, 0.0)
    base = s * RPT
    for m in range(RPT // K):
        if m * K < 400:
            pltpu.sync_copy(ones_v, acc.at[pl.ds(base + m * K, K)])
        else:
            @pl.when(s < NS - 1)
            def _():
                pltpu.sync_copy(ones_v, acc.at[pl.ds(base + m * K, K)])
    lax.fori_loop(0, K, fill, 1.0)
    plsc.subcore_barrier()

    def body(j, carry):
        pltpu.sync_copy(ones_v, acc.at[dst_v.at[j]], add=True)
        return carry

    lax.fori_loop(0, NJ, body, 0)
    plsc.subcore_barrier()

    @pl.when(s < NS - 1)
    def _():
        pltpu.sync_copy(acc.at[pl.ds(base, RPT)], out_hbm.at[c, pl.ds(base, RPT)])

    @pl.when(s == NS - 1)
    def _():
        pltpu.sync_copy(acc.at[pl.ds(N - 400, 400)],
                        out_hbm.at[c, pl.ds(N - 400, 400)])


@functools.partial(
    pl.kernel,
    out_type=jax.ShapeDtypeStruct((NC, N, D_H), jnp.float32),
    mesh=_mesh,
    compiler_params=_sc_params,
    scratch_types=[
        pltpu.VMEM((NJ, K), jnp.int32),
        pltpu.VMEM((NJ, K), jnp.int32),
        pltpu.VMEM((K, D_H), jnp.float32),
        pltpu.VMEM_SHARED((N, D_H), jnp.float32),
    ],
)
def _edge_kernel(hs_hbm, src_hbm, dst_hbm, out_hbm, src_v, dst_v, rows_v, acc):
    c = lax.axis_index("c")
    s = lax.axis_index("s")
    w = s * NC + c

    pltpu.sync_copy(src_hbm.at[w], src_v)
    pltpu.sync_copy(dst_hbm.at[w], dst_v)

    zero = jnp.zeros((L,), jnp.float32)

    def zbody(i, carry):
        def zcol(k2, carry2):
            rows_v[i, pl.ds(k2 * L, L)] = zero
            return carry2

        return lax.fori_loop(0, D_H // L, zcol, carry)

    lax.fori_loop(0, K, zbody, 0)

    base = s * RPT
    for m in range(RPT // K):
        if m * K < 400:
            pltpu.sync_copy(rows_v, acc.at[pl.ds(base + m * K, K)])
        else:
            @pl.when(s < NS - 1)
            def _():
                pltpu.sync_copy(rows_v, acc.at[pl.ds(base + m * K, K)])
    plsc.subcore_barrier()

    def body(j, carry):
        pltpu.sync_copy(hs_hbm.at[src_v.at[j]], rows_v)
        pltpu.sync_copy(rows_v, acc.at[dst_v.at[j]], add=True)
        return carry

    lax.fori_loop(0, NJ, body, 0)
    plsc.subcore_barrier()

    @pl.when(s < NS - 1)
    def _():
        pltpu.sync_copy(acc.at[pl.ds(base, RPT)], out_hbm.at[c, pl.ds(base, RPT)])

    @pl.when(s == NS - 1)
    def _():
        pltpu.sync_copy(acc.at[pl.ds(N - 400, 400)],
                        out_hbm.at[c, pl.ds(N - 400, 400)])


def _tc_first_body(hist_ref, x_ref, w0_ref, dinv_ref, hs_ref):
    deg = hist_ref[0, :, 0:1] + hist_ref[1, :, 0:1] + 1.0
    dinv = lax.rsqrt(deg)
    h = jnp.dot(x_ref[...], w0_ref[...], preferred_element_type=jnp.float32)
    dinv_ref[...] = dinv
    hs_ref[...] = dinv * h


def _tc_mid_body(acc_ref, hs_ref, dinv_ref, b_ref, gw_ref, gb_ref, ga_ref,
                 wn_ref, hsn_ref):
    dinv = dinv_ref[...]
    sacc = acc_ref[0] + acc_ref[1] + hs_ref[...]
    conv = dinv * sacc + b_ref[...]
    mean = jnp.mean(conv, axis=0, keepdims=True)
    xc = conv - ga_ref[...] * mean
    var = jnp.mean(xc * xc, axis=0, keepdims=True)
    g = gw_ref[...] * xc * lax.rsqrt(var + 1e-5) + gb_ref[...]
    hsn_ref[...] = dinv * jnp.dot(g, wn_ref[...],
                                  preferred_element_type=jnp.float32)


def _tc_last_body(acc_ref, hs_ref, dinv_ref, b_ref, gw_ref, gb_ref, ga_ref,
                  out_ref):
    sacc = acc_ref[0] + acc_ref[1] + hs_ref[...]
    conv = dinv_ref[...] * sacc + b_ref[...]
    mean = jnp.mean(conv, axis=0, keepdims=True)
    xc = conv - ga_ref[...] * mean
    var = jnp.mean(xc * xc, axis=0, keepdims=True)
    out_ref[...] = gw_ref[...] * xc * lax.rsqrt(var + 1e-5) + gb_ref[...]


_f32 = jnp.float32
_tc_first = pl.pallas_call(
    _tc_first_body,
    out_shape=[jax.ShapeDtypeStruct((N, 1), _f32),
               jax.ShapeDtypeStruct((N, D_H), _f32)],
)
_tc_mid = pl.pallas_call(
    _tc_mid_body,
    out_shape=jax.ShapeDtypeStruct((N, D_H), _f32),
)
_tc_last = pl.pallas_call(
    _tc_last_body,
    out_shape=jax.ShapeDtypeStruct((N, D_H), _f32),
)


def kernel(x, edge_index, W0, b0, gw0, gb0, ga0, W1, b1, gw1, gb1, ga1,
           W2, b2, gw2, gb2, ga2):
    src_r = edge_index[0].reshape(NW, NJ, K)
    dst_r = edge_index[1].reshape(NW, NJ, K)

    histp = _deg_kernel(dst_r)
    dinv, hs = _tc_first(histp, x, W0)

    params = [(b0, gw0, gb0, ga0), (b1, gw1, gb1, ga1), (b2, gw2, gb2, ga2)]
    row = lambda v: v.reshape(1, D_H)

    for layer in range(3):
        acc = _edge_kernel(hs, src_r, dst_r)
        b, gw, gb, ga = (row(v) for v in params[layer])
        if layer < 2:
            wn = (W1, W2)[layer]
            hs = _tc_mid(acc, hs, dinv, b, gw, gb, ga, wn)
        else:
            out = _tc_last(acc, hs, dinv, b, gw, gb, ga)
    return out


# speedup vs baseline: 23.5741x; 1.1840x over previous
import functools

import jax
import jax.numpy as jnp
from jax import lax
from jax.experimental import pallas as pl
from jax.experimental.pallas import tpu as pltpu
from jax.experimental.pallas import tpu_sc as plsc

N = 10000
E = 320000
D_IN = 128
D_H = 64

NC = 2
NS = 16
NW = NC * NS
EPW = E // NW
K = 80
NJ = EPW // K
RPT = 640
L = 16

_mesh = plsc.VectorSubcoreMesh(core_axis_name="c", subcore_axis_name="s")
_sc_params = pltpu.CompilerParams(use_tc_tiling_on_sc=False)


@functools.partial(
    pl.kernel,
    out_type=jax.ShapeDtypeStruct((NC, N, L), jnp.float32),
    mesh=_mesh,
    compiler_params=_sc_params,
    scratch_types=[
        pltpu.VMEM((NJ, K), jnp.int32),
        pltpu.VMEM((K, L), jnp.float32),
        pltpu.VMEM_SHARED((N, L), jnp.float32),
    ],
)
def _deg_kernel(dst_hbm, out_hbm, dst_v, ones_v, acc):
    c = lax.axis_index("c")
    s = lax.axis_index("s")
    w = s * NC + c
    pltpu.sync_copy(dst_hbm.at[w], dst_v)

    def fill(i, carry):
        ones_v[i, :] = jnp.full((L,), carry, jnp.float32)
        return carry

    lax.fori_loop(0, K, fill, 0.0)
    base = s * RPT
    for m in range(RPT // K):
        if m * K < 400:
            pltpu.sync_copy(ones_v, acc.at[pl.ds(base + m * K, K)])
        else:
            @pl.when(s < NS - 1)
            def _():
                pltpu.sync_copy(ones_v, acc.at[pl.ds(base + m * K, K)])
    lax.fori_loop(0, K, fill, 1.0)
    plsc.subcore_barrier()

    def body(j, carry):
        pltpu.sync_copy(ones_v, acc.at[dst_v.at[j]], add=True)
        return carry

    lax.fori_loop(0, NJ, body, 0)
    plsc.subcore_barrier()

    @pl.when(s < NS - 1)
    def _():
        pltpu.sync_copy(acc.at[pl.ds(base, RPT)], out_hbm.at[c, pl.ds(base, RPT)])

    @pl.when(s == NS - 1)
    def _():
        pltpu.sync_copy(acc.at[pl.ds(N - 400, 400)],
                        out_hbm.at[c, pl.ds(N - 400, 400)])


@functools.partial(
    pl.kernel,
    out_type=jax.ShapeDtypeStruct((NC, N, D_H), jnp.float32),
    mesh=_mesh,
    compiler_params=_sc_params,
    scratch_types=[
        pltpu.VMEM((NJ, K), jnp.int32),
        pltpu.VMEM((NJ, K), jnp.int32),
        pltpu.VMEM((K, D_H), jnp.float32),
        pltpu.VMEM((K, D_H), jnp.float32),
        pltpu.VMEM_SHARED((N, D_H), jnp.float32),
        pltpu.SemaphoreType.DMA,
        pltpu.SemaphoreType.DMA,
    ],
)
def _edge_kernel(hs_hbm, src_hbm, dst_hbm, out_hbm, src_v, dst_v, rows_v,
                 rows2_v, acc, sem0, sem1):
    c = lax.axis_index("c")
    s = lax.axis_index("s")
    w = s * NC + c

    pltpu.sync_copy(src_hbm.at[w], src_v)
    pltpu.sync_copy(dst_hbm.at[w], dst_v)

    zero = jnp.zeros((L,), jnp.float32)

    def zbody(i, carry):
        def zcol(k2, carry2):
            rows_v[i, pl.ds(k2 * L, L)] = zero
            return carry2

        return lax.fori_loop(0, D_H // L, zcol, carry)

    lax.fori_loop(0, K, zbody, 0)

    base = s * RPT
    for m in range(RPT // K):
        if m * K < 400:
            pltpu.sync_copy(rows_v, acc.at[pl.ds(base + m * K, K)])
        else:
            @pl.when(s < NS - 1)
            def _():
                pltpu.sync_copy(rows_v, acc.at[pl.ds(base + m * K, K)])
    plsc.subcore_barrier()

    pltpu.async_copy(hs_hbm.at[src_v.at[0]], rows_v, sem0)

    def body(i, carry):
        j = 2 * i
        pltpu.make_async_copy(hs_hbm.at[src_v.at[j]], rows_v, sem0).wait()
        pltpu.async_copy(hs_hbm.at[src_v.at[j + 1]], rows2_v, sem1)
        pltpu.sync_copy(rows_v, acc.at[dst_v.at[j]], add=True)
        pltpu.make_async_copy(hs_hbm.at[src_v.at[j + 1]], rows2_v, sem1).wait()
        pltpu.async_copy(hs_hbm.at[src_v.at[j + 2]], rows_v, sem0)
        pltpu.sync_copy(rows2_v, acc.at[dst_v.at[j + 1]], add=True)
        return carry

    lax.fori_loop(0, (NJ - 1) // 2, body, 0)
    pltpu.make_async_copy(hs_hbm.at[src_v.at[NJ - 1]], rows_v, sem0).wait()
    pltpu.sync_copy(rows_v, acc.at[dst_v.at[NJ - 1]], add=True)
    plsc.subcore_barrier()

    @pl.when(s < NS - 1)
    def _():
        pltpu.sync_copy(acc.at[pl.ds(base, RPT)], out_hbm.at[c, pl.ds(base, RPT)])

    @pl.when(s == NS - 1)
    def _():
        pltpu.sync_copy(acc.at[pl.ds(N - 400, 400)],
                        out_hbm.at[c, pl.ds(N - 400, 400)])


def _tc_first_body(hist_ref, x_ref, w0_ref, dinv_ref, hs_ref):
    deg = hist_ref[0, :, 0:1] + hist_ref[1, :, 0:1] + 1.0
    dinv = lax.rsqrt(deg)
    h = jnp.dot(x_ref[...], w0_ref[...], preferred_element_type=jnp.float32)
    dinv_ref[...] = dinv
    hs_ref[...] = dinv * h


def _tc_mid_body(acc_ref, hs_ref, dinv_ref, b_ref, gw_ref, gb_ref, ga_ref,
                 wn_ref, hsn_ref):
    dinv = dinv_ref[...]
    sacc = acc_ref[0] + acc_ref[1] + hs_ref[...]
    conv = dinv * sacc + b_ref[...]
    mean = jnp.mean(conv, axis=0, keepdims=True)
    xc = conv - ga_ref[...] * mean
    var = jnp.mean(xc * xc, axis=0, keepdims=True)
    g = gw_ref[...] * xc * lax.rsqrt(var + 1e-5) + gb_ref[...]
    hsn_ref[...] = dinv * jnp.dot(g, wn_ref[...],
                                  preferred_element_type=jnp.float32)


def _tc_last_body(acc_ref, hs_ref, dinv_ref, b_ref, gw_ref, gb_ref, ga_ref,
                  out_ref):
    sacc = acc_ref[0] + acc_ref[1] + hs_ref[...]
    conv = dinv_ref[...] * sacc + b_ref[...]
    mean = jnp.mean(conv, axis=0, keepdims=True)
    xc = conv - ga_ref[...] * mean
    var = jnp.mean(xc * xc, axis=0, keepdims=True)
    out_ref[...] = gw_ref[...] * xc * lax.rsqrt(var + 1e-5) + gb_ref[...]


_f32 = jnp.float32
_tc_first = pl.pallas_call(
    _tc_first_body,
    out_shape=[jax.ShapeDtypeStruct((N, 1), _f32),
               jax.ShapeDtypeStruct((N, D_H), _f32)],
)
_tc_mid = pl.pallas_call(
    _tc_mid_body,
    out_shape=jax.ShapeDtypeStruct((N, D_H), _f32),
)
_tc_last = pl.pallas_call(
    _tc_last_body,
    out_shape=jax.ShapeDtypeStruct((N, D_H), _f32),
)


def kernel(x, edge_index, W0, b0, gw0, gb0, ga0, W1, b1, gw1, gb1, ga1,
           W2, b2, gw2, gb2, ga2):
    src_r = edge_index[0].reshape(NW, NJ, K)
    dst_r = edge_index[1].reshape(NW, NJ, K)

    histp = _deg_kernel(dst_r)
    dinv, hs = _tc_first(histp, x, W0)

    params = [(b0, gw0, gb0, ga0), (b1, gw1, gb1, ga1), (b2, gw2, gb2, ga2)]
    row = lambda v: v.reshape(1, D_H)

    for layer in range(3):
        acc = _edge_kernel(hs, src_r, dst_r)
        b, gw, gb, ga = (row(v) for v in params[layer])
        if layer < 2:
            wn = (W1, W2)[layer]
            hs = _tc_mid(acc, hs, dinv, b, gw, gb, ga, wn)
        else:
            out = _tc_last(acc, hs, dinv, b, gw, gb, ga)
    return out
